# Initial kernel scaffold; baseline (speedup 1.0000x reference)
import jax
import jax.numpy as jnp
from jax.experimental import pallas as pl


def kernel(inputs, targets):
    raise NotImplementedError("write your pallas kernel here")


# speedup vs baseline: 1.7046x; 1.7046x over previous
import functools

import jax
import jax.numpy as jnp
from jax import lax
from jax.experimental import pallas as pl

_MARGIN = 0.3
_B = 4096
_D = 512
_C = 256

_HI = lax.Precision.HIGHEST


def _dot0(a, b):
    return lax.dot_general(a, b, (((0,), (0,)), ((), ())), precision=_HI)


def _loss_body(x_ref, t_ref, out_ref):
    x = x_ref[...]
    t = t_ref[...]
    cix = lax.broadcasted_iota(jnp.int32, (1, _C), 1)
    onehot = (t == cix).astype(jnp.float32)

    ones_col = jnp.ones((_B, 1), jnp.float32)
    counts = _dot0(onehot, ones_col)
    sums = _dot0(onehot, x)
    avg = sums / jnp.maximum(counts, 1.0)
    present = (counts > 0.0).astype(jnp.float32)
    p_total = jnp.sum(present)
    s_row = jnp.sum(avg, axis=0, keepdims=True)

    a2 = jnp.sum(avg * avg, axis=1, keepdims=True)
    sdot = jnp.sum(avg * s_row, axis=1, keepdims=True)
    s2 = jnp.sum(s_row * s_row)
    denom = p_total - present
    restn2 = (s2 - 2.0 * sdot + a2) / (denom * denom)

    rowi = lax.broadcasted_iota(jnp.int32, (_C, _C), 0)
    coli = lax.broadcasted_iota(jnp.int32, (_C, _C), 1)
    lower = (coli < rowi).astype(jnp.float32)
    rank = lax.dot_general(lower, present, (((1,), (0,)), ((), ())),
                           precision=_HI)

    g = lax.dot_general(x, avg, (((1,), (1,)), ((), ())),
                        precision=_HI)
    xs2 = jnp.sum(x * x, axis=1, keepdims=True)
    x_dot_s = jnp.sum(g, axis=1, keepdims=True)
    g_t = jnp.sum(g * onehot, axis=1, keepdims=True)
    a2_t = lax.dot_general(onehot, a2, (((1,), (0,)), ((), ())),
                           precision=_HI)
    r = lax.dot_general(onehot, rank, (((1,), (0,)), ((), ())),
                        precision=_HI)
    cix_f = cix.astype(jnp.float32)
    onehot_r = (r == cix_f).astype(jnp.float32)
    g_r = jnp.sum(g * onehot_r, axis=1, keepdims=True)
    denom_r = lax.dot_general(onehot_r, denom, (((1,), (0,)), ((), ())),
                              precision=_HI)
    restn2_r = lax.dot_general(onehot_r, restn2, (((1,), (0,)), ((), ())),
                               precision=_HI)

    dap = jnp.sqrt(jnp.maximum(xs2 - 2.0 * g_t + a2_t, 0.0))
    dan = jnp.sqrt(jnp.maximum(
        xs2 - 2.0 * (x_dot_s - g_r) / denom_r + restn2_r, 0.0))
    loss = jnp.sum(jnp.maximum(0.0, dap - dan + _MARGIN),
                   axis=0, keepdims=True) * (1.0 / _B)
    out_ref[...] = loss


@jax.jit
def kernel(inputs, targets):
    t2 = targets.astype(jnp.int32).reshape(_B, 1)
    out = pl.pallas_call(
        _loss_body,
        out_shape=jax.ShapeDtypeStruct((1, 1), jnp.float32),
    )(inputs, t2)
    return out[0, 0]
